# baseline (speedup 1.0000x reference)
import functools

import jax
import jax.numpy as jnp
from jax import lax
from jax.experimental import pallas as pl
from jax.experimental.pallas import tpu as pltpu
from jax.experimental.pallas import tpu_sc as plsc

_NC = 2
_NS = 16
_NW = _NC * _NS
_CH = 128


def _silu(x):
    return x * jax.nn.sigmoid(x)


def _gather_call(E, N, D):
    nchunks = E // _CH
    nfull = nchunks // _NW
    extra = nchunks % _NW

    def body(row_r, col_r, h_r, coord_r,
             hrow_o, hcol_o, diff4_o,
             coord_v, ridx0, cidx0, hrow0, hcol0, diff0,
             ridx1, cidx1, hrow1, hcol1, diff1, semG0, semG1, semW0, semW1):
        c = lax.axis_index("c")
        s = lax.axis_index("s")
        wid = s * _NC + c
        pltpu.sync_copy(coord_r, coord_v)
        nch = nfull + jnp.where(wid < extra, 1, 0)
        bufs = ((ridx0, cidx0, hrow0, hcol0, diff0, semG0, semW0),
                (ridx1, cidx1, hrow1, hcol1, diff1, semG1, semW1))

        def cwork(ri, ci, dv):
            def one(j, carry2):
                r3 = ri[pl.ds(j * 16, 16)] * 3
                c3 = ci[pl.ds(j * 16, 16)] * 3
                dx = plsc.load_gather(coord_v, [r3]) - plsc.load_gather(coord_v, [c3])
                dy = plsc.load_gather(coord_v, [r3 + 1]) - plsc.load_gather(coord_v, [c3 + 1])
                dz = plsc.load_gather(coord_v, [r3 + 2]) - plsc.load_gather(coord_v, [c3 + 2])
                r2 = dx * dx + dy * dy + dz * dz
                rows4 = (j * 16 + lax.iota(jnp.int32, 16)) * 4
                plsc.store_scatter(dv, [rows4], dx)
                plsc.store_scatter(dv, [rows4 + 1], dy)
                plsc.store_scatter(dv, [rows4 + 2], dz)
                plsc.store_scatter(dv, [rows4 + 3], r2)
                return carry2
            lax.fori_loop(0, _CH // 16, one, 0)

        def finish(i, p):
            ri, ci, hr, hc, dv, semG, semW = bufs[p]
            base = (wid + i * _NW) * _CH
            pltpu.make_async_copy(h_r.at[ri], hr, semG).wait()
            pltpu.make_async_copy(h_r.at[ci], hc, semG).wait()
            cwork(ri, ci, dv)
            pltpu.async_copy(hr, hrow_o.at[pl.ds(base, _CH)], semW)
            pltpu.async_copy(hc, hcol_o.at[pl.ds(base, _CH)], semW)
            pltpu.async_copy(dv, diff4_o.at[pl.ds(base * 4, _CH * 4)], semW)

        def drainW(p):
            ri, ci, hr, hc, dv, semG, semW = bufs[p]
            pltpu.make_async_copy(hr, hrow_o.at[pl.ds(0, _CH)], semW).wait()
            pltpu.make_async_copy(hc, hcol_o.at[pl.ds(0, _CH)], semW).wait()
            pltpu.make_async_copy(dv, diff4_o.at[pl.ds(0, _CH * 4)], semW).wait()

        def chunk(i, carry):
            base = (wid + i * _NW) * _CH
            for p in range(2):
                ri, ci, hr, hc, dv, semG, semW = bufs[p]

                @pl.when(i % 2 == p)
                def _():
                    @pl.when(i >= 2)
                    def _():
                        drainW(p)

                    pltpu.sync_copy(row_r.at[pl.ds(base, _CH)], ri)
                    pltpu.sync_copy(col_r.at[pl.ds(base, _CH)], ci)
                    pltpu.async_copy(h_r.at[ri], hr, semG)
                    pltpu.async_copy(h_r.at[ci], hc, semG)

                    @pl.when(i >= 1)
                    def _():
                        finish(i - 1, 1 - p)

            return carry

        lax.fori_loop(0, nch, chunk, 0)

        for p in range(2):
            @pl.when((nch - 1) % 2 == p)
            def _():
                finish(nch - 1, p)

        for p in range(2):
            @pl.when(nch >= 2 - p)
            def _():
                drainW(p)

    mesh = plsc.VectorSubcoreMesh(core_axis_name="c", subcore_axis_name="s")
    return pl.kernel(
        body,
        out_type=(
            jax.ShapeDtypeStruct((E, D), jnp.float32),
            jax.ShapeDtypeStruct((E, D), jnp.float32),
            jax.ShapeDtypeStruct((E * 4,), jnp.float32),
        ),
        mesh=mesh,
        compiler_params=pltpu.CompilerParams(needs_layout_passes=False),
        scratch_types=[
            pltpu.VMEM((N * 3,), jnp.float32),
            pltpu.VMEM((_CH,), jnp.int32),
            pltpu.VMEM((_CH,), jnp.int32),
            pltpu.VMEM((_CH, D), jnp.float32),
            pltpu.VMEM((_CH, D), jnp.float32),
            pltpu.VMEM((_CH * 4,), jnp.float32),
            pltpu.VMEM((_CH,), jnp.int32),
            pltpu.VMEM((_CH,), jnp.int32),
            pltpu.VMEM((_CH, D), jnp.float32),
            pltpu.VMEM((_CH, D), jnp.float32),
            pltpu.VMEM((_CH * 4,), jnp.float32),
            pltpu.SemaphoreType.DMA,
            pltpu.SemaphoreType.DMA,
            pltpu.SemaphoreType.DMA,
            pltpu.SemaphoreType.DMA,
        ],
    )


def _edge_call(E, D, H, B):
    grid = (E // B,)

    def body(hrow, hcol, diff4, w1a, w1b, w1c, b1, w2, b2, cw1, cb1, cw2,
             ef_o, gate_o):
        d4 = diff4[...]
        r2 = d4[:, 3:4]
        pre1 = jnp.dot(hrow[...], w1a[...], preferred_element_type=jnp.float32)
        pre1 = pre1 + jnp.dot(hcol[...], w1b[...], preferred_element_type=jnp.float32)
        pre1 = pre1 + r2 * w1c[...] + b1[...]
        t1 = _silu(pre1)
        f = _silu(jnp.dot(t1, w2[...], preferred_element_type=jnp.float32) + b2[...])
        g1 = _silu(jnp.dot(f, cw1[...], preferred_element_type=jnp.float32) + cb1[...])
        gate = jnp.dot(g1, cw2[...], preferred_element_type=jnp.float32)
        ef_o[...] = f
        gate_o[...] = gate

    full2 = lambda shape: pl.BlockSpec(shape, lambda i: (0, 0))
    full1 = lambda shape: pl.BlockSpec(shape, lambda i: (0,))
    return pl.pallas_call(
        body,
        grid=grid,
        in_specs=[
            pl.BlockSpec((B, D), lambda i: (i, 0)),
            pl.BlockSpec((B, D), lambda i: (i, 0)),
            pl.BlockSpec((B, 4), lambda i: (i, 0)),
            full2((D, H)), full2((D, H)), full2((1, H)), full1((H,)),
            full2((H, H)), full1((H,)),
            full2((H, H)), full1((H,)), full2((H, 1)),
        ],
        out_specs=[
            pl.BlockSpec((B, H), lambda i: (i, 0)),
            pl.BlockSpec((B, 1), lambda i: (i, 0)),
        ],
        out_shape=[
            jax.ShapeDtypeStruct((E, H), jnp.float32),
            jax.ShapeDtypeStruct((E, 1), jnp.float32),
        ],
    )


def _scatter_call(E, N, H):
    nchunks = E // _CH
    nfull = nchunks // _NW
    extra = nchunks % _NW
    RPT = -(-(N + _NS - 1) // _NS // 8) * 8
    NP = _NS * RPT

    NSTAGE = RPT // _CH
    TAILR = RPT - NSTAGE * _CH

    def body(row_r, ef_r,
             pf_o,
             accf, ridx_v, ef_v, ridx2_v, ef2_v, semA, semB):
        c = lax.axis_index("c")
        s = lax.axis_index("s")
        wid = s * _NC + c
        r0 = s * RPT

        def z1(i, carry):
            ef_v[i // 8, pl.ds((i % 8) * 16, 16)] = jnp.zeros((16,), jnp.float32)
            return carry
        lax.fori_loop(0, _CH * 8, z1, 0)

        for k in range(NSTAGE):
            pltpu.sync_copy(ef_v, accf.at[pl.ds(r0 + k * _CH, _CH)])
        pltpu.sync_copy(ef_v.at[pl.ds(0, TAILR)], accf.at[pl.ds(r0 + NSTAGE * _CH, TAILR)])
        plsc.subcore_barrier()

        nch = nfull + jnp.where(wid < extra, 1, 0)
        bufs = ((ridx_v, ef_v, semA), (ridx2_v, ef2_v, semB))

        pltpu.async_copy(row_r.at[pl.ds(wid * _CH, _CH)], ridx_v, semA)
        pltpu.async_copy(ef_r.at[pl.ds(wid * _CH, _CH)], ef_v, semA)

        def chunk(i, carry):
            base = (wid + i * _NW) * _CH
            nbase = base + _NW * _CH

            for p in range(2):
                ri, efv, sem = bufs[p]
                rn, efn, semn = bufs[1 - p]

                @pl.when(i % 2 == p)
                def _():
                    pltpu.make_async_copy(row_r.at[pl.ds(base, _CH)], ri, sem).wait()
                    pltpu.make_async_copy(ef_r.at[pl.ds(base, _CH)], efv, sem).wait()

                    @pl.when(i + 1 < nch)
                    def _():
                        pltpu.async_copy(row_r.at[pl.ds(nbase, _CH)], rn, semn)
                        pltpu.async_copy(ef_r.at[pl.ds(nbase, _CH)], efn, semn)

                    pltpu.sync_copy(efv, accf.at[ri], add=True)

            return carry

        lax.fori_loop(0, nch, chunk, 0)
        plsc.subcore_barrier()

        for k in range(NSTAGE):
            pltpu.sync_copy(accf.at[pl.ds(r0 + k * _CH, _CH)], ef_v)
            pltpu.sync_copy(ef_v, pf_o.at[pl.ds(c * NP + r0 + k * _CH, _CH)])
        pltpu.sync_copy(accf.at[pl.ds(r0 + NSTAGE * _CH, TAILR)], ef_v.at[pl.ds(0, TAILR)])
        pltpu.sync_copy(ef_v.at[pl.ds(0, TAILR)], pf_o.at[pl.ds(c * NP + r0 + NSTAGE * _CH, TAILR)])

    mesh = plsc.VectorSubcoreMesh(core_axis_name="c", subcore_axis_name="s")
    return pl.kernel(
        body,
        out_type=(
            jax.ShapeDtypeStruct((_NC * NP, H), jnp.float32),
        ),
        mesh=mesh,
        compiler_params=pltpu.CompilerParams(needs_layout_passes=False),
        scratch_types=[
            pltpu.VMEM_SHARED((NP, H), jnp.float32),
            pltpu.VMEM((_CH,), jnp.int32),
            pltpu.VMEM((_CH, H), jnp.float32),
            pltpu.VMEM((_CH,), jnp.int32),
            pltpu.VMEM((_CH, H), jnp.float32),
            pltpu.SemaphoreType.DMA,
            pltpu.SemaphoreType.DMA,
        ],
    )


def _scatter_tr_call(E, N, H):
    nchunks = E // _CH
    nfull = nchunks // _NW
    extra = nchunks % _NW
    RPT = -(-(N + _NS - 1) // _NS // 8) * 8
    NP = _NS * RPT
    NSTAGE = RPT // _CH
    TAILR = RPT - NSTAGE * _CH

    def body(row_r, gate_r, diff_r,
             pt_o,
             acct, pay_v, ridx0, gate0, diff0, ridx1, gate1, diff1, semA, semB):
        c = lax.axis_index("c")
        s = lax.axis_index("s")
        wid = s * _NC + c
        r0 = s * RPT

        def z1(i, carry):
            pay_v[i // 8, pl.ds((i % 8) * 16, 16)] = jnp.zeros((16,), jnp.float32)
            return carry
        lax.fori_loop(0, _CH * 8, z1, 0)
        for k in range(NSTAGE):
            pltpu.sync_copy(pay_v, acct.at[pl.ds(r0 + k * _CH, _CH)])
        pltpu.sync_copy(pay_v.at[pl.ds(0, TAILR)], acct.at[pl.ds(r0 + NSTAGE * _CH, TAILR)])
        plsc.subcore_barrier()

        head = jnp.where(lax.iota(jnp.int32, 16) == 3, 1.0, 0.0)

        def z2(r, carry):
            pay_v[r, pl.ds(0, 16)] = head
            return carry
        lax.fori_loop(0, _CH, z2, 0)

        nch = nfull + jnp.where(wid < extra, 1, 0)
        bufs = ((ridx0, gate0, diff0, semA), (ridx1, gate1, diff1, semB))

        pltpu.async_copy(row_r.at[pl.ds(wid * _CH, _CH)], ridx0, semA)
        pltpu.async_copy(gate_r.at[pl.ds(wid * _CH, _CH)], gate0, semA)
        pltpu.async_copy(diff_r.at[pl.ds(wid * _CH * 4, _CH * 4)], diff0, semA)

        def chunk(i, carry):
            base = (wid + i * _NW) * _CH
            nbase = base + _NW * _CH

            for p in range(2):
                ri, gv, dv, sem = bufs[p]
                rn, gn, dn, semn = bufs[1 - p]

                @pl.when(i % 2 == p)
                def _():
                    pltpu.make_async_copy(row_r.at[pl.ds(base, _CH)], ri, sem).wait()
                    pltpu.make_async_copy(gate_r.at[pl.ds(base, _CH)], gv, sem).wait()
                    pltpu.make_async_copy(diff_r.at[pl.ds(base * 4, _CH * 4)], dv, sem).wait()

                    @pl.when(i + 1 < nch)
                    def _():
                        pltpu.async_copy(row_r.at[pl.ds(nbase, _CH)], rn, semn)
                        pltpu.async_copy(gate_r.at[pl.ds(nbase, _CH)], gn, semn)
                        pltpu.async_copy(diff_r.at[pl.ds(nbase * 4, _CH * 4)], dn, semn)

                    @plsc.parallel_loop(0, _CH // 16, 1, unroll=2)
                    def _(j):
                        e4 = (j * 16 + lax.iota(jnp.int32, 16)) * 4
                        g = gv[pl.ds(j * 16, 16)]
                        tx = plsc.load_gather(dv, [e4]) * g
                        ty = plsc.load_gather(dv, [e4 + 1]) * g
                        tz = plsc.load_gather(dv, [e4 + 2]) * g
                        rows = j * 16 + lax.iota(jnp.int32, 16)
                        plsc.store_scatter(pay_v, [rows, jnp.zeros((16,), jnp.int32)], tx)
                        plsc.store_scatter(pay_v, [rows, jnp.full((16,), 1, jnp.int32)], ty)
                        plsc.store_scatter(pay_v, [rows, jnp.full((16,), 2, jnp.int32)], tz)

                    pltpu.sync_copy(pay_v, acct.at[ri], add=True)

            return carry

        lax.fori_loop(0, nch, chunk, 0)
        plsc.subcore_barrier()

        for k in range(NSTAGE):
            pltpu.sync_copy(acct.at[pl.ds(r0 + k * _CH, _CH)], pay_v)
            pltpu.sync_copy(pay_v, pt_o.at[pl.ds(c * NP + r0 + k * _CH, _CH)])
        pltpu.sync_copy(acct.at[pl.ds(r0 + NSTAGE * _CH, TAILR)], pay_v.at[pl.ds(0, TAILR)])
        pltpu.sync_copy(pay_v.at[pl.ds(0, TAILR)], pt_o.at[pl.ds(c * NP + r0 + NSTAGE * _CH, TAILR)])

    mesh = plsc.VectorSubcoreMesh(core_axis_name="c", subcore_axis_name="s")
    return pl.kernel(
        body,
        out_type=(
            jax.ShapeDtypeStruct((_NC * NP, H), jnp.float32),
        ),
        mesh=mesh,
        compiler_params=pltpu.CompilerParams(needs_layout_passes=False),
        scratch_types=[
            pltpu.VMEM_SHARED((NP, H), jnp.float32),
            pltpu.VMEM((_CH, H), jnp.float32),
            pltpu.VMEM((_CH,), jnp.int32),
            pltpu.VMEM((_CH,), jnp.float32),
            pltpu.VMEM((_CH * 4,), jnp.float32),
            pltpu.VMEM((_CH,), jnp.int32),
            pltpu.VMEM((_CH,), jnp.float32),
            pltpu.VMEM((_CH * 4,), jnp.float32),
            pltpu.SemaphoreType.DMA,
            pltpu.SemaphoreType.DMA,
        ],
    )


def _node_call(N, D, H, B):
    grid = (N // B,)

    def body(h, pf, p4, coord, nw1a, nw1b, nb1, nw2, nb2, h_o, c_o):
        nagg = pf[0] + pf[1]
        hh = h[...]
        pre = jnp.dot(hh, nw1a[...], preferred_element_type=jnp.float32)
        pre = pre + jnp.dot(nagg, nw1b[...], preferred_element_type=jnp.float32)
        pre = pre + nb1[...]
        out = jnp.dot(_silu(pre), nw2[...], preferred_element_type=jnp.float32) + nb2[...]
        h_o[...] = hh + out
        t4 = p4[0] + p4[1]
        cnt = t4[:, 3:4]
        c_o[...] = coord[...] + t4[:, 0:3] / jnp.maximum(cnt, 1.0)

    full2 = lambda shape: pl.BlockSpec(shape, lambda i: (0, 0))
    full1 = lambda shape: pl.BlockSpec(shape, lambda i: (0,))
    return pl.pallas_call(
        body,
        grid=grid,
        in_specs=[
            pl.BlockSpec((B, D), lambda i: (i, 0)),
            pl.BlockSpec((_NC, B, H), lambda i: (0, i, 0)),
            pl.BlockSpec((_NC, B, 16), lambda i: (0, i, 0)),
            pl.BlockSpec((B, 3), lambda i: (i, 0)),
            full2((D, H)), full2((H, H)), full1((H,)),
            full2((H, H)), full1((H,)),
        ],
        out_specs=[
            pl.BlockSpec((B, H), lambda i: (i, 0)),
            pl.BlockSpec((B, 3), lambda i: (i, 0)),
        ],
        out_shape=[
            jax.ShapeDtypeStruct((N, H), jnp.float32),
            jax.ShapeDtypeStruct((N, 3), jnp.float32),
        ],
    )


def kernel(h, edge_index, coord, edge_w1, edge_b1, edge_w2, edge_b2,
           coord_w1, coord_b1, coord_w2, node_w1, node_b1, node_w2, node_b2):
    N, D = h.shape
    E = edge_index.shape[1]
    H = edge_w2.shape[0]
    row = edge_index[0]
    col = edge_index[1]

    hrow, hcol, diff4f = _gather_call(E, N, D)(row, col, h, coord.reshape(-1))
    diff4 = diff4f.reshape(E, 4)

    w1a = edge_w1[:D]
    w1b = edge_w1[D:2 * D]
    w1c = edge_w1[2 * D:2 * D + 1]
    ef, gatec = _edge_call(E, D, H, 2000)(
        hrow, hcol, diff4, w1a, w1b, w1c, edge_b1, edge_w2, edge_b2,
        coord_w1, coord_b1, coord_w2)

    NP = 16 * (-(-(N + 15) // 16 // 8) * 8)
    (pfp,) = _scatter_call(E, N, H)(row, ef)
    (ptp,) = _scatter_tr_call(E, N, H)(row, gatec.reshape(E), diff4f)
    pf = pfp.reshape(2, NP, H)[:, :N]
    p16 = ptp.reshape(2, NP, H)[:, :N, :16]

    nw1a = node_w1[:D]
    nw1b = node_w1[D:]
    h_out, coord_out = _node_call(N, D, H, 1000)(
        h, pf, p16, coord, nw1a, nw1b, node_b1, node_w2, node_b2)
    return (h_out, coord_out)


# speedup vs baseline: 1.0065x; 1.0065x over previous
import functools

import jax
import jax.numpy as jnp
from jax import lax
from jax.experimental import pallas as pl
from jax.experimental.pallas import tpu as pltpu
from jax.experimental.pallas import tpu_sc as plsc

_NC = 2
_NS = 16
_NW = _NC * _NS
_CH = 128


def _silu(x):
    return x * jax.nn.sigmoid(x)


def _gather_call(E, N, D):
    nchunks = E // _CH
    nfull = nchunks // _NW
    extra = nchunks % _NW

    def body(row_r, col_r, h_r, coord_r,
             hrow_o, hcol_o, diff4_o,
             coord_v, ridx0, cidx0, hrow0, hcol0, diff0,
             ridx1, cidx1, hrow1, hcol1, diff1, semG0, semG1, semW0, semW1):
        c = lax.axis_index("c")
        s = lax.axis_index("s")
        wid = s * _NC + c
        pltpu.sync_copy(coord_r, coord_v)
        nch = nfull + jnp.where(wid < extra, 1, 0)
        bufs = ((ridx0, cidx0, hrow0, hcol0, diff0, semG0, semW0),
                (ridx1, cidx1, hrow1, hcol1, diff1, semG1, semW1))

        def cwork(ri, ci, dv):
            def one(j, carry2):
                r3 = ri[pl.ds(j * 16, 16)] * 3
                c3 = ci[pl.ds(j * 16, 16)] * 3
                dx = plsc.load_gather(coord_v, [r3]) - plsc.load_gather(coord_v, [c3])
                dy = plsc.load_gather(coord_v, [r3 + 1]) - plsc.load_gather(coord_v, [c3 + 1])
                dz = plsc.load_gather(coord_v, [r3 + 2]) - plsc.load_gather(coord_v, [c3 + 2])
                r2 = dx * dx + dy * dy + dz * dz
                rows4 = (j * 16 + lax.iota(jnp.int32, 16)) * 4
                plsc.store_scatter(dv, [rows4], dx)
                plsc.store_scatter(dv, [rows4 + 1], dy)
                plsc.store_scatter(dv, [rows4 + 2], dz)
                plsc.store_scatter(dv, [rows4 + 3], r2)
                return carry2
            lax.fori_loop(0, _CH // 16, one, 0)

        def finish(i, p):
            ri, ci, hr, hc, dv, semG, semW = bufs[p]
            base = (wid + i * _NW) * _CH
            pltpu.make_async_copy(h_r.at[ri], hr, semG).wait()
            pltpu.make_async_copy(h_r.at[ci], hc, semG).wait()
            cwork(ri, ci, dv)
            pltpu.async_copy(hr, hrow_o.at[pl.ds(base, _CH)], semW)
            pltpu.async_copy(hc, hcol_o.at[pl.ds(base, _CH)], semW)
            pltpu.async_copy(dv, diff4_o.at[pl.ds(base * 4, _CH * 4)], semW)

        def drainW(p):
            ri, ci, hr, hc, dv, semG, semW = bufs[p]
            pltpu.make_async_copy(hr, hrow_o.at[pl.ds(0, _CH)], semW).wait()
            pltpu.make_async_copy(hc, hcol_o.at[pl.ds(0, _CH)], semW).wait()
            pltpu.make_async_copy(dv, diff4_o.at[pl.ds(0, _CH * 4)], semW).wait()

        def chunk(i, carry):
            base = (wid + i * _NW) * _CH
            for p in range(2):
                ri, ci, hr, hc, dv, semG, semW = bufs[p]

                @pl.when(i % 2 == p)
                def _():
                    @pl.when(i >= 2)
                    def _():
                        drainW(p)

                    pltpu.sync_copy(row_r.at[pl.ds(base, _CH)], ri)
                    pltpu.sync_copy(col_r.at[pl.ds(base, _CH)], ci)
                    pltpu.async_copy(h_r.at[ri], hr, semG)
                    pltpu.async_copy(h_r.at[ci], hc, semG)

                    @pl.when(i >= 1)
                    def _():
                        finish(i - 1, 1 - p)

            return carry

        lax.fori_loop(0, nch, chunk, 0)

        for p in range(2):
            @pl.when((nch - 1) % 2 == p)
            def _():
                finish(nch - 1, p)

        for p in range(2):
            @pl.when(nch >= 2 - p)
            def _():
                drainW(p)

    mesh = plsc.VectorSubcoreMesh(core_axis_name="c", subcore_axis_name="s")
    return pl.kernel(
        body,
        out_type=(
            jax.ShapeDtypeStruct((E, D), jnp.float32),
            jax.ShapeDtypeStruct((E, D), jnp.float32),
            jax.ShapeDtypeStruct((E * 4,), jnp.float32),
        ),
        mesh=mesh,
        compiler_params=pltpu.CompilerParams(needs_layout_passes=False),
        scratch_types=[
            pltpu.VMEM((N * 3,), jnp.float32),
            pltpu.VMEM((_CH,), jnp.int32),
            pltpu.VMEM((_CH,), jnp.int32),
            pltpu.VMEM((_CH, D), jnp.float32),
            pltpu.VMEM((_CH, D), jnp.float32),
            pltpu.VMEM((_CH * 4,), jnp.float32),
            pltpu.VMEM((_CH,), jnp.int32),
            pltpu.VMEM((_CH,), jnp.int32),
            pltpu.VMEM((_CH, D), jnp.float32),
            pltpu.VMEM((_CH, D), jnp.float32),
            pltpu.VMEM((_CH * 4,), jnp.float32),
            pltpu.SemaphoreType.DMA,
            pltpu.SemaphoreType.DMA,
            pltpu.SemaphoreType.DMA,
            pltpu.SemaphoreType.DMA,
        ],
    )


def _edge_call(E, D, H, B):
    grid = (E // B,)

    def body(hrow, hcol, diff4, w1a, w1b, w1c, b1, w2, b2, cw1, cb1, cw2,
             ef_o, tr128_o):
        d4 = diff4[...]
        r2 = d4[:, 3:4]
        pre1 = jnp.dot(hrow[...], w1a[...], preferred_element_type=jnp.float32)
        pre1 = pre1 + jnp.dot(hcol[...], w1b[...], preferred_element_type=jnp.float32)
        pre1 = pre1 + r2 * w1c[...] + b1[...]
        t1 = _silu(pre1)
        f = _silu(jnp.dot(t1, w2[...], preferred_element_type=jnp.float32) + b2[...])
        g1 = _silu(jnp.dot(f, cw1[...], preferred_element_type=jnp.float32) + cb1[...])
        gate = jnp.dot(g1, cw2[...], preferred_element_type=jnp.float32)
        lane = lax.broadcasted_iota(jnp.int32, (B, H), 1)
        d128 = jnp.pad(d4, ((0, 0), (0, H - 4)))
        tr128 = jnp.where(lane == 3, 1.0, d128 * gate)
        ef_o[...] = f
        tr128_o[...] = tr128

    full2 = lambda shape: pl.BlockSpec(shape, lambda i: (0, 0))
    full1 = lambda shape: pl.BlockSpec(shape, lambda i: (0,))
    return pl.pallas_call(
        body,
        grid=grid,
        in_specs=[
            pl.BlockSpec((B, D), lambda i: (i, 0)),
            pl.BlockSpec((B, D), lambda i: (i, 0)),
            pl.BlockSpec((B, 4), lambda i: (i, 0)),
            full2((D, H)), full2((D, H)), full2((1, H)), full1((H,)),
            full2((H, H)), full1((H,)),
            full2((H, H)), full1((H,)), full2((H, 1)),
        ],
        out_specs=[
            pl.BlockSpec((B, H), lambda i: (i, 0)),
            pl.BlockSpec((B, H), lambda i: (i, 0)),
        ],
        out_shape=[
            jax.ShapeDtypeStruct((E, H), jnp.float32),
            jax.ShapeDtypeStruct((E, H), jnp.float32),
        ],
    )


def _scatter2_call(E, N, H):
    nchunks = E // _CH
    nfull = nchunks // _NW
    extra = nchunks % _NW
    RPT = -(-(N + _NS - 1) // _NS // 8) * 8
    NP = _NS * RPT
    NSTAGE = RPT // _CH
    TAILR = RPT - NSTAGE * _CH

    def body(row_r, ef_r, tr_r,
             pf_o, pt_o,
             accf, ridx_v, ef_v, ridx2_v, ef2_v, semA, semB):
        c = lax.axis_index("c")
        s = lax.axis_index("s")
        wid = s * _NC + c
        r0 = s * RPT
        nch = nfull + jnp.where(wid < extra, 1, 0)
        bufs = ((ridx_v, ef_v, semA), (ridx2_v, ef2_v, semB))

        def onepass(pay_r, out_o):
            def z1(i, carry):
                ef_v[i // 8, pl.ds((i % 8) * 16, 16)] = jnp.zeros((16,), jnp.float32)
                return carry
            lax.fori_loop(0, _CH * 8, z1, 0)
            for k in range(NSTAGE):
                pltpu.sync_copy(ef_v, accf.at[pl.ds(r0 + k * _CH, _CH)])
            pltpu.sync_copy(ef_v.at[pl.ds(0, TAILR)], accf.at[pl.ds(r0 + NSTAGE * _CH, TAILR)])
            plsc.subcore_barrier()

            pltpu.async_copy(row_r.at[pl.ds(wid * _CH, _CH)], ridx_v, semA)
            pltpu.async_copy(pay_r.at[pl.ds(wid * _CH, _CH)], ef_v, semA)

            def chunk(i, carry):
                base = (wid + i * _NW) * _CH
                nbase = base + _NW * _CH
                for p in range(2):
                    ri, efv, sem = bufs[p]
                    rn, efn, semn = bufs[1 - p]

                    @pl.when(i % 2 == p)
                    def _():
                        pltpu.make_async_copy(row_r.at[pl.ds(base, _CH)], ri, sem).wait()
                        pltpu.make_async_copy(pay_r.at[pl.ds(base, _CH)], efv, sem).wait()

                        @pl.when(i + 1 < nch)
                        def _():
                            pltpu.async_copy(row_r.at[pl.ds(nbase, _CH)], rn, semn)
                            pltpu.async_copy(pay_r.at[pl.ds(nbase, _CH)], efn, semn)

                        pltpu.sync_copy(efv, accf.at[ri], add=True)

                return carry

            lax.fori_loop(0, nch, chunk, 0)
            plsc.subcore_barrier()

            for k in range(NSTAGE):
                pltpu.sync_copy(accf.at[pl.ds(r0 + k * _CH, _CH)], ef_v)
                pltpu.sync_copy(ef_v, out_o.at[pl.ds(c * NP + r0 + k * _CH, _CH)])
            pltpu.sync_copy(accf.at[pl.ds(r0 + NSTAGE * _CH, TAILR)], ef_v.at[pl.ds(0, TAILR)])
            pltpu.sync_copy(ef_v.at[pl.ds(0, TAILR)], out_o.at[pl.ds(c * NP + r0 + NSTAGE * _CH, TAILR)])
            plsc.subcore_barrier()

        onepass(ef_r, pf_o)
        onepass(tr_r, pt_o)

    mesh = plsc.VectorSubcoreMesh(core_axis_name="c", subcore_axis_name="s")
    return pl.kernel(
        body,
        out_type=(
            jax.ShapeDtypeStruct((_NC * NP, H), jnp.float32),
            jax.ShapeDtypeStruct((_NC * NP, H), jnp.float32),
        ),
        mesh=mesh,
        compiler_params=pltpu.CompilerParams(needs_layout_passes=False),
        scratch_types=[
            pltpu.VMEM_SHARED((NP, H), jnp.float32),
            pltpu.VMEM((_CH,), jnp.int32),
            pltpu.VMEM((_CH, H), jnp.float32),
            pltpu.VMEM((_CH,), jnp.int32),
            pltpu.VMEM((_CH, H), jnp.float32),
            pltpu.SemaphoreType.DMA,
            pltpu.SemaphoreType.DMA,
        ],
    )


def _node_call(N, D, H, B):
    grid = (N // B,)

    def body(h, pf, p4, coord, nw1a, nw1b, nb1, nw2, nb2, h_o, c_o):
        nagg = pf[0] + pf[1]
        hh = h[...]
        pre = jnp.dot(hh, nw1a[...], preferred_element_type=jnp.float32)
        pre = pre + jnp.dot(nagg, nw1b[...], preferred_element_type=jnp.float32)
        pre = pre + nb1[...]
        out = jnp.dot(_silu(pre), nw2[...], preferred_element_type=jnp.float32) + nb2[...]
        h_o[...] = hh + out
        t4 = p4[0] + p4[1]
        cnt = t4[:, 3:4]
        c_o[...] = coord[...] + t4[:, 0:3] / jnp.maximum(cnt, 1.0)

    full2 = lambda shape: pl.BlockSpec(shape, lambda i: (0, 0))
    full1 = lambda shape: pl.BlockSpec(shape, lambda i: (0,))
    return pl.pallas_call(
        body,
        grid=grid,
        in_specs=[
            pl.BlockSpec((B, D), lambda i: (i, 0)),
            pl.BlockSpec((_NC, B, H), lambda i: (0, i, 0)),
            pl.BlockSpec((_NC, B, 16), lambda i: (0, i, 0)),
            pl.BlockSpec((B, 3), lambda i: (i, 0)),
            full2((D, H)), full2((H, H)), full1((H,)),
            full2((H, H)), full1((H,)),
        ],
        out_specs=[
            pl.BlockSpec((B, H), lambda i: (i, 0)),
            pl.BlockSpec((B, 3), lambda i: (i, 0)),
        ],
        out_shape=[
            jax.ShapeDtypeStruct((N, H), jnp.float32),
            jax.ShapeDtypeStruct((N, 3), jnp.float32),
        ],
    )


def kernel(h, edge_index, coord, edge_w1, edge_b1, edge_w2, edge_b2,
           coord_w1, coord_b1, coord_w2, node_w1, node_b1, node_w2, node_b2):
    N, D = h.shape
    E = edge_index.shape[1]
    H = edge_w2.shape[0]
    row = edge_index[0]
    col = edge_index[1]

    hrow, hcol, diff4f = _gather_call(E, N, D)(row, col, h, coord.reshape(-1))
    diff4 = diff4f.reshape(E, 4)

    w1a = edge_w1[:D]
    w1b = edge_w1[D:2 * D]
    w1c = edge_w1[2 * D:2 * D + 1]
    ef, tr128 = _edge_call(E, D, H, 2000)(
        hrow, hcol, diff4, w1a, w1b, w1c, edge_b1, edge_w2, edge_b2,
        coord_w1, coord_b1, coord_w2)

    NP = 16 * (-(-(N + 15) // 16 // 8) * 8)
    pfp, ptp = _scatter2_call(E, N, H)(row, ef, tr128)
    pf = pfp.reshape(2, NP, H)[:, :N]
    p16 = ptp.reshape(2, NP, H)[:, :N, :16]

    nw1a = node_w1[:D]
    nw1b = node_w1[D:]
    h_out, coord_out = _node_call(N, D, H, 1000)(
        h, pf, p16, coord, nw1a, nw1b, node_b1, node_w2, node_b2)
    return (h_out, coord_out)


# speedup vs baseline: 1.0647x; 1.0579x over previous
import functools

import jax
import jax.numpy as jnp
from jax import lax
from jax.experimental import pallas as pl
from jax.experimental.pallas import tpu as pltpu
from jax.experimental.pallas import tpu_sc as plsc

_NC = 2
_NS = 16
_NW = _NC * _NS
_CH = 128


def _silu(x):
    return x * jax.nn.sigmoid(x)


def _gather_call(E, N, D):
    nchunks = E // _CH
    nfull = nchunks // _NW
    extra = nchunks % _NW

    def body(row_r, col_r, h_r, coord_r,
             hrow_o, hcol_o, diff4_o,
             coord_v, ridx0, cidx0, hrow0, hcol0, diff0,
             ridx1, cidx1, hrow1, hcol1, diff1, semG0, semG1, semW0, semW1):
        c = lax.axis_index("c")
        s = lax.axis_index("s")
        wid = s * _NC + c
        pltpu.sync_copy(coord_r, coord_v)
        nch = nfull + jnp.where(wid < extra, 1, 0)
        bufs = ((ridx0, cidx0, hrow0, hcol0, diff0, semG0, semW0),
                (ridx1, cidx1, hrow1, hcol1, diff1, semG1, semW1))

        def cwork(ri, ci, dv):
            def one(j, carry2):
                r3 = ri[pl.ds(j * 16, 16)] * 3
                c3 = ci[pl.ds(j * 16, 16)] * 3
                dx = plsc.load_gather(coord_v, [r3]) - plsc.load_gather(coord_v, [c3])
                dy = plsc.load_gather(coord_v, [r3 + 1]) - plsc.load_gather(coord_v, [c3 + 1])
                dz = plsc.load_gather(coord_v, [r3 + 2]) - plsc.load_gather(coord_v, [c3 + 2])
                r2 = dx * dx + dy * dy + dz * dz
                rows4 = (j * 16 + lax.iota(jnp.int32, 16)) * 4
                plsc.store_scatter(dv, [rows4], dx)
                plsc.store_scatter(dv, [rows4 + 1], dy)
                plsc.store_scatter(dv, [rows4 + 2], dz)
                plsc.store_scatter(dv, [rows4 + 3], r2)
                return carry2
            lax.fori_loop(0, _CH // 16, one, 0)

        def finish(i, p):
            ri, ci, hr, hc, dv, semG, semW = bufs[p]
            base = (wid + i * _NW) * _CH
            pltpu.make_async_copy(h_r.at[ri], hr, semG).wait()
            pltpu.make_async_copy(h_r.at[ci], hc, semG).wait()
            cwork(ri, ci, dv)
            pltpu.async_copy(hr, hrow_o.at[pl.ds(base, _CH)], semW)
            pltpu.async_copy(hc, hcol_o.at[pl.ds(base, _CH)], semW)
            pltpu.async_copy(dv, diff4_o.at[pl.ds(base * 4, _CH * 4)], semW)

        def drainW(p):
            ri, ci, hr, hc, dv, semG, semW = bufs[p]
            pltpu.make_async_copy(hr, hrow_o.at[pl.ds(0, _CH)], semW).wait()
            pltpu.make_async_copy(hc, hcol_o.at[pl.ds(0, _CH)], semW).wait()
            pltpu.make_async_copy(dv, diff4_o.at[pl.ds(0, _CH * 4)], semW).wait()

        def chunk(i, carry):
            base = (wid + i * _NW) * _CH
            for p in range(2):
                ri, ci, hr, hc, dv, semG, semW = bufs[p]

                @pl.when(i % 2 == p)
                def _():
                    @pl.when(i >= 2)
                    def _():
                        drainW(p)

                    pltpu.sync_copy(row_r.at[pl.ds(base, _CH)], ri)
                    pltpu.sync_copy(col_r.at[pl.ds(base, _CH)], ci)
                    pltpu.async_copy(h_r.at[ri], hr, semG)
                    pltpu.async_copy(h_r.at[ci], hc, semG)

                    @pl.when(i >= 1)
                    def _():
                        finish(i - 1, 1 - p)

            return carry

        lax.fori_loop(0, nch, chunk, 0)

        for p in range(2):
            @pl.when((nch - 1) % 2 == p)
            def _():
                finish(nch - 1, p)

        for p in range(2):
            @pl.when(nch >= 2 - p)
            def _():
                drainW(p)

    mesh = plsc.VectorSubcoreMesh(core_axis_name="c", subcore_axis_name="s")
    return pl.kernel(
        body,
        out_type=(
            jax.ShapeDtypeStruct((E, D), jnp.float32),
            jax.ShapeDtypeStruct((E, D), jnp.float32),
            jax.ShapeDtypeStruct((E * 4,), jnp.float32),
        ),
        mesh=mesh,
        compiler_params=pltpu.CompilerParams(needs_layout_passes=False),
        scratch_types=[
            pltpu.VMEM((N * 3,), jnp.float32),
            pltpu.VMEM((_CH,), jnp.int32),
            pltpu.VMEM((_CH,), jnp.int32),
            pltpu.VMEM((_CH, D), jnp.float32),
            pltpu.VMEM((_CH, D), jnp.float32),
            pltpu.VMEM((_CH * 4,), jnp.float32),
            pltpu.VMEM((_CH,), jnp.int32),
            pltpu.VMEM((_CH,), jnp.int32),
            pltpu.VMEM((_CH, D), jnp.float32),
            pltpu.VMEM((_CH, D), jnp.float32),
            pltpu.VMEM((_CH * 4,), jnp.float32),
            pltpu.SemaphoreType.DMA,
            pltpu.SemaphoreType.DMA,
            pltpu.SemaphoreType.DMA,
            pltpu.SemaphoreType.DMA,
        ],
    )


def _edge_call(E, D, H, B):
    grid = (E // B,)

    def body(hrow, hcol, diff4, w1a, w1b, w1c, b1, w2, b2, cw1, cb1, cw2,
             ef_o, tr128_o):
        d4 = diff4[...]
        r2 = d4[:, 3:4]
        pre1 = jnp.dot(hrow[...], w1a[...], preferred_element_type=jnp.float32)
        pre1 = pre1 + jnp.dot(hcol[...], w1b[...], preferred_element_type=jnp.float32)
        pre1 = pre1 + r2 * w1c[...] + b1[...]
        t1 = _silu(pre1)
        f = _silu(jnp.dot(t1, w2[...], preferred_element_type=jnp.float32) + b2[...])
        g1 = _silu(jnp.dot(f, cw1[...], preferred_element_type=jnp.float32) + cb1[...])
        gate = jnp.dot(g1, cw2[...], preferred_element_type=jnp.float32)
        lane = lax.broadcasted_iota(jnp.int32, (B, H), 1)
        d128 = jnp.pad(d4, ((0, 0), (0, H - 4)))
        tr128 = jnp.where(lane == 3, 1.0, d128 * gate)
        ef_o[...] = f
        tr128_o[...] = tr128

    full2 = lambda shape: pl.BlockSpec(shape, lambda i: (0, 0))
    full1 = lambda shape: pl.BlockSpec(shape, lambda i: (0,))
    return pl.pallas_call(
        body,
        grid=grid,
        in_specs=[
            pl.BlockSpec((B, D), lambda i: (i, 0)),
            pl.BlockSpec((B, D), lambda i: (i, 0)),
            pl.BlockSpec((B, 4), lambda i: (i, 0)),
            full2((D, H)), full2((D, H)), full2((1, H)), full1((H,)),
            full2((H, H)), full1((H,)),
            full2((H, H)), full1((H,)), full2((H, 1)),
        ],
        out_specs=[
            pl.BlockSpec((B, H), lambda i: (i, 0)),
            pl.BlockSpec((B, H), lambda i: (i, 0)),
        ],
        out_shape=[
            jax.ShapeDtypeStruct((E, H), jnp.float32),
            jax.ShapeDtypeStruct((E, H), jnp.float32),
        ],
    )


def _scatter2_call(E, N, H):
    nchunks = E // _CH
    nfull = nchunks // _NW
    extra = nchunks % _NW
    RPT = -(-(N + _NS - 1) // _NS // 8) * 8
    NP = _NS * RPT
    NSTAGE = RPT // _CH
    TAILR = RPT - NSTAGE * _CH

    def body(row_r, ef_r, tr_r,
             pf_o, pt_o,
             accf, ridx_v, ef_v, ridx2_v, ef2_v, semA, semB):
        c = lax.axis_index("c")
        s = lax.axis_index("s")
        wid = s * _NC + c
        r0 = s * RPT
        nch = nfull + jnp.where(wid < extra, 1, 0)
        bufs = ((ridx_v, ef_v, semA), (ridx2_v, ef2_v, semB))

        def onepass(pay_r, out_o):
            def z1(i, carry):
                ef_v[i // 8, pl.ds((i % 8) * 16, 16)] = jnp.zeros((16,), jnp.float32)
                return carry
            lax.fori_loop(0, _CH * 8, z1, 0)
            for k in range(NSTAGE):
                pltpu.sync_copy(ef_v, accf.at[pl.ds(r0 + k * _CH, _CH)])
            pltpu.sync_copy(ef_v.at[pl.ds(0, TAILR)], accf.at[pl.ds(r0 + NSTAGE * _CH, TAILR)])
            plsc.subcore_barrier()

            pltpu.async_copy(row_r.at[pl.ds(wid * _CH, _CH)], ridx_v, semA)
            pltpu.async_copy(pay_r.at[pl.ds(wid * _CH, _CH)], ef_v, semA)

            def chunk(i, carry):
                base = (wid + i * _NW) * _CH
                nbase = base + _NW * _CH
                for p in range(2):
                    ri, efv, sem = bufs[p]
                    rn, efn, semn = bufs[1 - p]

                    @pl.when(i % 2 == p)
                    def _():
                        pltpu.make_async_copy(row_r.at[pl.ds(base, _CH)], ri, sem).wait()
                        pltpu.make_async_copy(pay_r.at[pl.ds(base, _CH)], efv, sem).wait()

                        @pl.when(i + 1 < nch)
                        def _():
                            pltpu.async_copy(row_r.at[pl.ds(nbase, _CH)], rn, semn)
                            pltpu.async_copy(pay_r.at[pl.ds(nbase, _CH)], efn, semn)

                        pltpu.sync_copy(efv, accf.at[ri], add=True)

                return carry

            lax.fori_loop(0, nch, chunk, 0)
            plsc.subcore_barrier()

            for k in range(NSTAGE):
                pltpu.sync_copy(accf.at[pl.ds(r0 + k * _CH, _CH)], ef_v)
                pltpu.sync_copy(ef_v, out_o.at[pl.ds(c * NP + r0 + k * _CH, _CH)])
            pltpu.sync_copy(accf.at[pl.ds(r0 + NSTAGE * _CH, TAILR)], ef_v.at[pl.ds(0, TAILR)])
            pltpu.sync_copy(ef_v.at[pl.ds(0, TAILR)], out_o.at[pl.ds(c * NP + r0 + NSTAGE * _CH, TAILR)])
            plsc.subcore_barrier()

        onepass(ef_r, pf_o)
        onepass(tr_r, pt_o)

    mesh = plsc.VectorSubcoreMesh(core_axis_name="c", subcore_axis_name="s")
    return pl.kernel(
        body,
        out_type=(
            jax.ShapeDtypeStruct((_NC * NP, H), jnp.float32),
            jax.ShapeDtypeStruct((_NC * NP, H), jnp.float32),
        ),
        mesh=mesh,
        compiler_params=pltpu.CompilerParams(needs_layout_passes=False),
        scratch_types=[
            pltpu.VMEM_SHARED((NP, H), jnp.float32),
            pltpu.VMEM((_CH,), jnp.int32),
            pltpu.VMEM((_CH, H), jnp.float32),
            pltpu.VMEM((_CH,), jnp.int32),
            pltpu.VMEM((_CH, H), jnp.float32),
            pltpu.SemaphoreType.DMA,
            pltpu.SemaphoreType.DMA,
        ],
    )


def _node_call(N, D, H, B):
    grid = (N // B,)

    def body(h, pf, p4, coord, nw1a, nw1b, nb1, nw2, nb2, h_o, c_o):
        nagg = pf[0] + pf[1]
        hh = h[...]
        pre = jnp.dot(hh, nw1a[...], preferred_element_type=jnp.float32)
        pre = pre + jnp.dot(nagg, nw1b[...], preferred_element_type=jnp.float32)
        pre = pre + nb1[...]
        out = jnp.dot(_silu(pre), nw2[...], preferred_element_type=jnp.float32) + nb2[...]
        h_o[...] = hh + out
        t4 = p4[0] + p4[1]
        cnt = t4[:, 3:4]
        c_o[...] = coord[...] + t4[:, 0:3] / jnp.maximum(cnt, 1.0)

    full2 = lambda shape: pl.BlockSpec(shape, lambda i: (0, 0))
    full1 = lambda shape: pl.BlockSpec(shape, lambda i: (0,))
    return pl.pallas_call(
        body,
        grid=grid,
        in_specs=[
            pl.BlockSpec((B, D), lambda i: (i, 0)),
            pl.BlockSpec((_NC, B, H), lambda i: (0, i, 0)),
            pl.BlockSpec((_NC, B, 16), lambda i: (0, i, 0)),
            pl.BlockSpec((B, 3), lambda i: (i, 0)),
            full2((D, H)), full2((H, H)), full1((H,)),
            full2((H, H)), full1((H,)),
        ],
        out_specs=[
            pl.BlockSpec((B, H), lambda i: (i, 0)),
            pl.BlockSpec((B, 3), lambda i: (i, 0)),
        ],
        out_shape=[
            jax.ShapeDtypeStruct((N, H), jnp.float32),
            jax.ShapeDtypeStruct((N, 3), jnp.float32),
        ],
    )


def kernel(h, edge_index, coord, edge_w1, edge_b1, edge_w2, edge_b2,
           coord_w1, coord_b1, coord_w2, node_w1, node_b1, node_w2, node_b2):
    N, D = h.shape
    E = edge_index.shape[1]
    H = edge_w2.shape[0]
    row = edge_index[0]
    col = edge_index[1]

    hrow, hcol, diff4f = _gather_call(E, N, D)(row, col, h, coord.reshape(-1))
    diff4 = diff4f.reshape(E, 4)

    w1a = edge_w1[:D]
    w1b = edge_w1[D:2 * D]
    w1c = edge_w1[2 * D:2 * D + 1]
    ef, tr128 = _edge_call(E, D, H, 4000)(
        hrow, hcol, diff4, w1a, w1b, w1c, edge_b1, edge_w2, edge_b2,
        coord_w1, coord_b1, coord_w2)

    NP = 16 * (-(-(N + 15) // 16 // 8) * 8)
    pfp, ptp = _scatter2_call(E, N, H)(row, ef, tr128)
    pf = pfp.reshape(2, NP, H)[:, :N]
    p16 = ptp.reshape(2, NP, H)[:, :N, :16]

    nw1a = node_w1[:D]
    nw1b = node_w1[D:]
    h_out, coord_out = _node_call(N, D, H, 1000)(
        h, pf, p16, coord, nw1a, nw1b, node_b1, node_w2, node_b2)
    return (h_out, coord_out)


# speedup vs baseline: 1.0756x; 1.0102x over previous
import functools

import jax
import jax.numpy as jnp
from jax import lax
from jax.experimental import pallas as pl
from jax.experimental.pallas import tpu as pltpu
from jax.experimental.pallas import tpu_sc as plsc

_NC = 2
_NS = 16
_NW = _NC * _NS
_CH = 128


def _silu(x):
    return x * jax.nn.sigmoid(x)


def _gather_call(E, N, D):
    nchunks = E // _CH
    nfull = nchunks // _NW
    extra = nchunks % _NW

    def body(row_r, col_r, h_r, coord_r,
             hrow_o, hcol_o, diff4_o,
             coord_v, ridx0, cidx0, hrow0, hcol0, diff0,
             ridx1, cidx1, hrow1, hcol1, diff1, semG0, semG1, semW0, semW1):
        c = lax.axis_index("c")
        s = lax.axis_index("s")
        wid = s * _NC + c
        pltpu.sync_copy(coord_r, coord_v)
        nch = nfull + jnp.where(wid < extra, 1, 0)
        bufs = ((ridx0, cidx0, hrow0, hcol0, diff0, semG0, semW0),
                (ridx1, cidx1, hrow1, hcol1, diff1, semG1, semW1))

        def cwork(ri, ci, dv):
            def one(j, carry2):
                r3 = ri[pl.ds(j * 16, 16)] * 3
                c3 = ci[pl.ds(j * 16, 16)] * 3
                dx = plsc.load_gather(coord_v, [r3]) - plsc.load_gather(coord_v, [c3])
                dy = plsc.load_gather(coord_v, [r3 + 1]) - plsc.load_gather(coord_v, [c3 + 1])
                dz = plsc.load_gather(coord_v, [r3 + 2]) - plsc.load_gather(coord_v, [c3 + 2])
                r2 = dx * dx + dy * dy + dz * dz
                rows4 = (j * 16 + lax.iota(jnp.int32, 16)) * 4
                plsc.store_scatter(dv, [rows4], dx)
                plsc.store_scatter(dv, [rows4 + 1], dy)
                plsc.store_scatter(dv, [rows4 + 2], dz)
                plsc.store_scatter(dv, [rows4 + 3], r2)
                return carry2
            lax.fori_loop(0, _CH // 16, one, 0)

        def finish(i, p):
            ri, ci, hr, hc, dv, semG, semW = bufs[p]
            base = (wid + i * _NW) * _CH
            pltpu.make_async_copy(h_r.at[ri], hr, semG).wait()
            pltpu.make_async_copy(h_r.at[ci], hc, semG).wait()
            cwork(ri, ci, dv)
            pltpu.async_copy(hr, hrow_o.at[pl.ds(base, _CH)], semW)
            pltpu.async_copy(hc, hcol_o.at[pl.ds(base, _CH)], semW)
            pltpu.async_copy(dv, diff4_o.at[pl.ds(base * 4, _CH * 4)], semW)

        def drainW(p):
            ri, ci, hr, hc, dv, semG, semW = bufs[p]
            pltpu.make_async_copy(hr, hrow_o.at[pl.ds(0, _CH)], semW).wait()
            pltpu.make_async_copy(hc, hcol_o.at[pl.ds(0, _CH)], semW).wait()
            pltpu.make_async_copy(dv, diff4_o.at[pl.ds(0, _CH * 4)], semW).wait()

        def chunk(i, carry):
            base = (wid + i * _NW) * _CH
            for p in range(2):
                ri, ci, hr, hc, dv, semG, semW = bufs[p]

                @pl.when(i % 2 == p)
                def _():
                    @pl.when(i >= 2)
                    def _():
                        drainW(p)

                    pltpu.sync_copy(row_r.at[pl.ds(base, _CH)], ri)
                    pltpu.sync_copy(col_r.at[pl.ds(base, _CH)], ci)
                    pltpu.async_copy(h_r.at[ri], hr, semG)
                    pltpu.async_copy(h_r.at[ci], hc, semG)

                    @pl.when(i >= 1)
                    def _():
                        finish(i - 1, 1 - p)

            return carry

        lax.fori_loop(0, nch, chunk, 0)

        for p in range(2):
            @pl.when((nch - 1) % 2 == p)
            def _():
                finish(nch - 1, p)

        for p in range(2):
            @pl.when(nch >= 2 - p)
            def _():
                drainW(p)

    mesh = plsc.VectorSubcoreMesh(core_axis_name="c", subcore_axis_name="s")
    return pl.kernel(
        body,
        out_type=(
            jax.ShapeDtypeStruct((E, D), jnp.float32),
            jax.ShapeDtypeStruct((E, D), jnp.float32),
            jax.ShapeDtypeStruct((E * 4,), jnp.float32),
        ),
        mesh=mesh,
        compiler_params=pltpu.CompilerParams(needs_layout_passes=False),
        scratch_types=[
            pltpu.VMEM((N * 3,), jnp.float32),
            pltpu.VMEM((_CH,), jnp.int32),
            pltpu.VMEM((_CH,), jnp.int32),
            pltpu.VMEM((_CH, D), jnp.float32),
            pltpu.VMEM((_CH, D), jnp.float32),
            pltpu.VMEM((_CH * 4,), jnp.float32),
            pltpu.VMEM((_CH,), jnp.int32),
            pltpu.VMEM((_CH,), jnp.int32),
            pltpu.VMEM((_CH, D), jnp.float32),
            pltpu.VMEM((_CH, D), jnp.float32),
            pltpu.VMEM((_CH * 4,), jnp.float32),
            pltpu.SemaphoreType.DMA,
            pltpu.SemaphoreType.DMA,
            pltpu.SemaphoreType.DMA,
            pltpu.SemaphoreType.DMA,
        ],
    )


def _edge_call(E, D, H, B):
    grid = (E // B,)

    def body(hrow, hcol, diff4, w1a, w1b, w1c, b1, w2, b2, cw1, cb1, cw2,
             ef_o, tr128_o):
        d4 = diff4[...]
        r2 = d4[:, 3:4]
        pre1 = jnp.dot(hrow[...], w1a[...], preferred_element_type=jnp.float32)
        pre1 = pre1 + jnp.dot(hcol[...], w1b[...], preferred_element_type=jnp.float32)
        pre1 = pre1 + r2 * w1c[...] + b1[...]
        t1 = _silu(pre1)
        f = _silu(jnp.dot(t1, w2[...], preferred_element_type=jnp.float32) + b2[...])
        g1 = _silu(jnp.dot(f, cw1[...], preferred_element_type=jnp.float32) + cb1[...])
        gate = jnp.dot(g1, cw2[...], preferred_element_type=jnp.float32)
        lane = lax.broadcasted_iota(jnp.int32, (B, H), 1)
        d128 = jnp.pad(d4, ((0, 0), (0, H - 4)))
        tr128 = jnp.where(lane == 3, 1.0, d128 * gate)
        ef_o[...] = f
        tr128_o[...] = tr128

    full2 = lambda shape: pl.BlockSpec(shape, lambda i: (0, 0))
    full1 = lambda shape: pl.BlockSpec(shape, lambda i: (0,))
    return pl.pallas_call(
        body,
        grid=grid,
        in_specs=[
            pl.BlockSpec((B, D), lambda i: (i, 0)),
            pl.BlockSpec((B, D), lambda i: (i, 0)),
            pl.BlockSpec((B, 4), lambda i: (i, 0)),
            full2((D, H)), full2((D, H)), full2((1, H)), full1((H,)),
            full2((H, H)), full1((H,)),
            full2((H, H)), full1((H,)), full2((H, 1)),
        ],
        out_specs=[
            pl.BlockSpec((B, H), lambda i: (i, 0)),
            pl.BlockSpec((B, H), lambda i: (i, 0)),
        ],
        out_shape=[
            jax.ShapeDtypeStruct((E, H), jnp.float32),
            jax.ShapeDtypeStruct((E, H), jnp.float32),
        ],
    )


def _scatter2_call(E, N, H):
    nchunks = E // _CH
    nfull = nchunks // _NW
    extra = nchunks % _NW
    RPT = -(-(N + _NS - 1) // _NS // 8) * 8
    NP = _NS * RPT
    NSTAGE = RPT // _CH
    TAILR = RPT - NSTAGE * _CH

    def body(row_r, ef_r, tr_r,
             pf_o, pt_o,
             accf, ridx_v, ef_v, ridx2_v, ef2_v, semA, semB):
        c = lax.axis_index("c")
        s = lax.axis_index("s")
        wid = s * _NC + c
        r0 = s * RPT
        nch = nfull + jnp.where(wid < extra, 1, 0)
        bufs = ((ridx_v, ef_v, semA), (ridx2_v, ef2_v, semB))

        def onepass(pay_r, out_o):
            def z1(i, carry):
                ef_v[i // 8, pl.ds((i % 8) * 16, 16)] = jnp.zeros((16,), jnp.float32)
                return carry
            lax.fori_loop(0, _CH * 8, z1, 0)
            for k in range(NSTAGE):
                pltpu.sync_copy(ef_v, accf.at[pl.ds(r0 + k * _CH, _CH)])
            pltpu.sync_copy(ef_v.at[pl.ds(0, TAILR)], accf.at[pl.ds(r0 + NSTAGE * _CH, TAILR)])
            plsc.subcore_barrier()

            pltpu.async_copy(row_r.at[pl.ds(wid * _CH, _CH)], ridx_v, semA)
            pltpu.async_copy(pay_r.at[pl.ds(wid * _CH, _CH)], ef_v, semA)

            def chunk(i, carry):
                base = (wid + i * _NW) * _CH
                nbase = base + _NW * _CH
                for p in range(2):
                    ri, efv, sem = bufs[p]
                    rn, efn, semn = bufs[1 - p]

                    @pl.when(i % 2 == p)
                    def _():
                        pltpu.make_async_copy(row_r.at[pl.ds(base, _CH)], ri, sem).wait()
                        pltpu.make_async_copy(pay_r.at[pl.ds(base, _CH)], efv, sem).wait()

                        @pl.when(i + 1 < nch)
                        def _():
                            pltpu.async_copy(row_r.at[pl.ds(nbase, _CH)], rn, semn)
                            pltpu.async_copy(pay_r.at[pl.ds(nbase, _CH)], efn, semn)

                        pltpu.sync_copy(efv, accf.at[ri], add=True)

                return carry

            lax.fori_loop(0, nch, chunk, 0)
            plsc.subcore_barrier()

            for k in range(NSTAGE):
                pltpu.sync_copy(accf.at[pl.ds(r0 + k * _CH, _CH)], ef_v)
                pltpu.sync_copy(ef_v, out_o.at[pl.ds(c * NP + r0 + k * _CH, _CH)])
            pltpu.sync_copy(accf.at[pl.ds(r0 + NSTAGE * _CH, TAILR)], ef_v.at[pl.ds(0, TAILR)])
            pltpu.sync_copy(ef_v.at[pl.ds(0, TAILR)], out_o.at[pl.ds(c * NP + r0 + NSTAGE * _CH, TAILR)])
            plsc.subcore_barrier()

        onepass(ef_r, pf_o)
        onepass(tr_r, pt_o)

    mesh = plsc.VectorSubcoreMesh(core_axis_name="c", subcore_axis_name="s")
    return pl.kernel(
        body,
        out_type=(
            jax.ShapeDtypeStruct((_NC * NP, H), jnp.float32),
            jax.ShapeDtypeStruct((_NC * NP, H), jnp.float32),
        ),
        mesh=mesh,
        compiler_params=pltpu.CompilerParams(needs_layout_passes=False),
        scratch_types=[
            pltpu.VMEM_SHARED((NP, H), jnp.float32),
            pltpu.VMEM((_CH,), jnp.int32),
            pltpu.VMEM((_CH, H), jnp.float32),
            pltpu.VMEM((_CH,), jnp.int32),
            pltpu.VMEM((_CH, H), jnp.float32),
            pltpu.SemaphoreType.DMA,
            pltpu.SemaphoreType.DMA,
        ],
    )


def _node_call(N, D, H, B):
    grid = (N // B,)

    def body(h, pf, p4, coord, nw1a, nw1b, nb1, nw2, nb2, h_o, c_o):
        nagg = pf[0] + pf[1]
        hh = h[...]
        pre = jnp.dot(hh, nw1a[...], preferred_element_type=jnp.float32)
        pre = pre + jnp.dot(nagg, nw1b[...], preferred_element_type=jnp.float32)
        pre = pre + nb1[...]
        out = jnp.dot(_silu(pre), nw2[...], preferred_element_type=jnp.float32) + nb2[...]
        h_o[...] = hh + out
        t4 = p4[0] + p4[1]
        cnt = t4[:, 3:4]
        c_o[...] = coord[...] + t4[:, 0:3] / jnp.maximum(cnt, 1.0)

    full2 = lambda shape: pl.BlockSpec(shape, lambda i: (0, 0))
    full1 = lambda shape: pl.BlockSpec(shape, lambda i: (0,))
    return pl.pallas_call(
        body,
        grid=grid,
        in_specs=[
            pl.BlockSpec((B, D), lambda i: (i, 0)),
            pl.BlockSpec((_NC, B, H), lambda i: (0, i, 0)),
            pl.BlockSpec((_NC, B, 16), lambda i: (0, i, 0)),
            pl.BlockSpec((B, 3), lambda i: (i, 0)),
            full2((D, H)), full2((H, H)), full1((H,)),
            full2((H, H)), full1((H,)),
        ],
        out_specs=[
            pl.BlockSpec((B, H), lambda i: (i, 0)),
            pl.BlockSpec((B, 3), lambda i: (i, 0)),
        ],
        out_shape=[
            jax.ShapeDtypeStruct((N, H), jnp.float32),
            jax.ShapeDtypeStruct((N, 3), jnp.float32),
        ],
    )


def kernel(h, edge_index, coord, edge_w1, edge_b1, edge_w2, edge_b2,
           coord_w1, coord_b1, coord_w2, node_w1, node_b1, node_w2, node_b2):
    N, D = h.shape
    E = edge_index.shape[1]
    H = edge_w2.shape[0]
    row = edge_index[0]
    col = edge_index[1]

    hrow, hcol, diff4f = _gather_call(E, N, D)(row, col, h, coord.reshape(-1))
    diff4 = diff4f.reshape(E, 4)

    w1a = edge_w1[:D]
    w1b = edge_w1[D:2 * D]
    w1c = edge_w1[2 * D:2 * D + 1]
    ef, tr128 = _edge_call(E, D, H, 8000)(
        hrow, hcol, diff4, w1a, w1b, w1c, edge_b1, edge_w2, edge_b2,
        coord_w1, coord_b1, coord_w2)

    NP = 16 * (-(-(N + 15) // 16 // 8) * 8)
    pfp, ptp = _scatter2_call(E, N, H)(row, ef, tr128)
    pf = pfp.reshape(2, NP, H)[:, :N]
    p16 = ptp.reshape(2, NP, H)[:, :N, :16]

    nw1a = node_w1[:D]
    nw1b = node_w1[D:]
    h_out, coord_out = _node_call(N, D, H, 1000)(
        h, pf, p16, coord, nw1a, nw1b, node_b1, node_w2, node_b2)
    return (h_out, coord_out)
